# baseline (speedup 1.0000x reference)
import functools
import jax
import jax.numpy as jnp
from jax.experimental import pallas as pl
from jax.experimental.pallas import tpu as pltpu

_S_CHUNK = 5000
_DECAY = (50000.0 / 16.0) ** 2


def _tc_body(locs_ref, qw_ref, g_ref, w1_ref, w2_ref, w3_ref, v_ref):
    c_in = g_ref.shape[-1]
    c_out = v_ref.shape[-1]
    locs = locs_ref[0]
    h = jnp.sin(locs @ w1_ref[...])
    h = jnp.sin(h @ w2_ref[...])
    filt = h @ w3_ref[...]
    r2 = jnp.sum(locs * locs, axis=1, keepdims=True)
    bump = jnp.exp(1.0 - 1.0 / (1.0 - _DECAY * r2 * r2))
    a = bump * qw_ref[0]
    qi = jax.lax.broadcasted_iota(jnp.int32, (c_in, c_in * c_out), 0)
    qj = jax.lax.broadcasted_iota(jnp.int32, (c_in, c_in * c_out), 1)
    q_mat = (qj // c_out == qi).astype(jnp.float32)
    rj = jax.lax.broadcasted_iota(jnp.int32, (c_in * c_out, c_out), 0)
    rt = jax.lax.broadcasted_iota(jnp.int32, (c_in * c_out, c_out), 1)
    r_mat = (rj % c_out == rt).astype(jnp.float32)
    for b in range(g_ref.shape[0]):
        g = g_ref[b, 0, 0]
        p = (g @ q_mat) * filt
        v_ref[b, 0, 0] = (p @ r_mat) * a


def _dense_stage(locs_r, qw_r, g_r, w1, w2, w3, n_co, n_sb):
    b = g_r.shape[0]
    s = _S_CHUNK
    c_in = g_r.shape[-1]
    c_out = w3.shape[1] // c_in
    grid = (n_co, n_sb)
    return pl.pallas_call(
        _tc_body,
        grid=grid,
        in_specs=[
            pl.BlockSpec((1, s, 2), lambda co, sb: (co * n_sb + sb, 0, 0)),
            pl.BlockSpec((1, s, 1), lambda co, sb: (co * n_sb + sb, 0, 0)),
            pl.BlockSpec((b, 1, 1, s, c_in), lambda co, sb: (0, co, sb, 0, 0)),
            pl.BlockSpec(w1.shape, lambda co, sb: (0, 0)),
            pl.BlockSpec(w2.shape, lambda co, sb: (0, 0)),
            pl.BlockSpec(w3.shape, lambda co, sb: (0, 0)),
        ],
        out_specs=pl.BlockSpec(
            (b, 1, 1, s, c_out), lambda co, sb: (0, co, sb, 0, 0)
        ),
        out_shape=jax.ShapeDtypeStruct((b, n_co, n_sb, s, c_out), jnp.float32),
    )(locs_r, qw_r, g_r, w1, w2, w3)


def kernel(features, eval_locs, eval_indices, quad_weights, W1, W2, W3):
    b, c_in, p_in = features.shape
    nnz = eval_indices.shape[0]
    c_out = W3.shape[1] // c_in
    p_out = p_in
    seg = nnz // c_in
    n_sb = seg // _S_CHUNK
    idx_out = eval_indices[:, 0].astype(jnp.int32)
    idx_in = eval_indices[:, 1].astype(jnp.int32)

    g = jnp.take(features, idx_in, axis=2)
    g_r = g.reshape(b, c_in, n_sb, _S_CHUNK, c_in)
    qw_g = jnp.take(quad_weights, idx_in)

    locs_r = eval_locs.reshape(c_in * n_sb, _S_CHUNK, 2)
    qw_r = qw_g.reshape(c_in * n_sb, _S_CHUNK, 1)

    v = _dense_stage(locs_r, qw_r, g_r, W1, W2, W3, c_in, n_sb)
    values2 = v.reshape(b, c_out, nnz)

    out = jnp.zeros((b, c_out, p_out), jnp.float32)
    out = out.at[:, :, idx_out].add(values2)
    return out


# speedup vs baseline: 19.5266x; 19.5266x over previous
import functools
import jax
import jax.numpy as jnp
from jax import lax
from jax.experimental import pallas as pl
from jax.experimental.pallas import tpu as pltpu
from jax.experimental.pallas import tpu_sc as plsc

_S_CHUNK = 5000
_DECAY = (50000.0 / 16.0) ** 2
_GCH = 8000
_QCH = 8000


def _sc_gather(features, quad_weights, idx_in):
    b, c, p = features.shape
    nnz = idx_in.shape[0]
    half = nnz // 2
    qper = nnz // 25
    mesh = plsc.VectorSubcoreMesh(core_axis_name="c", subcore_axis_name="s")

    @functools.partial(
        pl.kernel,
        mesh=mesh,
        compiler_params=pltpu.CompilerParams(needs_layout_passes=False),
        out_type=(
            jax.ShapeDtypeStruct((b * c * nnz,), jnp.float32),
            jax.ShapeDtypeStruct((nnz,), jnp.float32),
        ),
        scratch_types=[
            pltpu.VMEM((p,), jnp.float32),
            pltpu.VMEM((_GCH,), jnp.int32),
            pltpu.VMEM((_GCH,), jnp.float32),
        ],
    )
    def gk(feat_hbm, qw_hbm, idx_hbm, g_hbm, qwg_hbm, table_v, idx_v, out_v):
        wid = lax.axis_index("c") * 16 + lax.axis_index("s")
        plane = wid // 2
        base = (wid % 2) * half
        pltpu.sync_copy(feat_hbm.at[pl.ds(plane * p, p)], table_v)

        def chunk(k, _):
            off = base + k * _GCH

            def vec(j, _):
                iv = idx_v[pl.ds(j * 16, 16)]
                out_v[pl.ds(j * 16, 16)] = plsc.load_gather(table_v, [iv])
                return 0

            pltpu.sync_copy(idx_hbm.at[pl.ds(off, _GCH)], idx_v)
            lax.fori_loop(0, _GCH // 16, vec, 0)
            pltpu.sync_copy(
                out_v, g_hbm.at[pl.ds(plane * nnz + off, _GCH)]
            )
            return 0

        lax.fori_loop(0, half // _GCH, chunk, 0)

        @pl.when(wid < 25)
        def _():
            pltpu.sync_copy(qw_hbm, table_v)

            def qchunk(k, _):
                off = wid * qper + k * _QCH

                def vec(j, _):
                    iv = idx_v[pl.ds(j * 16, 16)]
                    out_v[pl.ds(j * 16, 16)] = plsc.load_gather(
                        table_v, [iv]
                    )
                    return 0

                pltpu.sync_copy(idx_hbm.at[pl.ds(off, _QCH)], idx_v)
                lax.fori_loop(0, _QCH // 16, vec, 0)
                pltpu.sync_copy(out_v, qwg_hbm.at[pl.ds(off, _QCH)])
                return 0

            lax.fori_loop(0, qper // _QCH, qchunk, 0)

    g_flat, qw_g = gk(features.reshape(-1), quad_weights, idx_in)
    return g_flat.reshape(b, c, nnz), qw_g


_SCH = 8000
_CCH = 10000


def _sc_scatter(values2, idx_out, p_out):
    b, c, nnz = values2.shape
    half = nnz // 2
    mesh = plsc.VectorSubcoreMesh(core_axis_name="c", subcore_axis_name="s")

    @functools.partial(
        pl.kernel,
        mesh=mesh,
        compiler_params=pltpu.CompilerParams(needs_layout_passes=False),
        out_type=jax.ShapeDtypeStruct((b * c * p_out,), jnp.float32),
        scratch_types=[
            pltpu.VMEM((p_out,), jnp.float32),
            pltpu.VMEM((_SCH,), jnp.int32),
            pltpu.VMEM((_SCH,), jnp.float32),
            pltpu.VMEM((_CCH,), jnp.float32),
            pltpu.VMEM_SHARED((8 * p_out,), jnp.float32),
        ],
    )
    def sk(val_hbm, idx_hbm, out_hbm, acc_v, idx_v, val_v, tmp_v, shared):
        cid = lax.axis_index("c")
        wid = cid * 16 + lax.axis_index("s")
        plane = wid // 2
        hh = wid % 2
        pl_loc = plane - cid * 8
        base = hh * half

        def zero(j, _):
            acc_v[pl.ds(j * 16, 16)] = jnp.zeros((16,), jnp.float32)
            return 0

        lax.fori_loop(0, p_out // 16, zero, 0)

        def chunk(k, _):
            off = base + k * _SCH
            pltpu.sync_copy(idx_hbm.at[pl.ds(off, _SCH)], idx_v)
            pltpu.sync_copy(
                val_hbm.at[pl.ds(plane * nnz + off, _SCH)], val_v
            )

            def vec(j, _):
                iv = idx_v[pl.ds(j * 16, 16)]
                vv = val_v[pl.ds(j * 16, 16)]
                plsc.addupdate_scatter(acc_v, [iv], vv)
                return 0

            lax.fori_loop(0, _SCH // 16, vec, 0)
            return 0

        lax.fori_loop(0, half // _SCH, chunk, 0)

        @pl.when(hh == 0)
        def _():
            pltpu.sync_copy(acc_v, shared.at[pl.ds(pl_loc * p_out, p_out)])

        plsc.subcore_barrier()

        @pl.when(hh == 1)
        def _():
            def comb(k, _):
                off = k * _CCH
                pltpu.sync_copy(
                    shared.at[pl.ds(pl_loc * p_out + off, _CCH)], tmp_v
                )

                def add(j, _):
                    sl = pl.ds(off + j * 16, 16)
                    acc_v[sl] = acc_v[sl] + tmp_v[pl.ds(j * 16, 16)]
                    return 0

                lax.fori_loop(0, _CCH // 16, add, 0)
                return 0

            lax.fori_loop(0, p_out // _CCH, comb, 0)
            pltpu.sync_copy(acc_v, out_hbm.at[pl.ds(plane * p_out, p_out)])

    return sk(values2.reshape(-1), idx_out).reshape(b, c, p_out)


def _tc_body(locs_ref, qw_ref, g_ref, w1_ref, w2_ref, w3_ref, v_ref):
    c_in = g_ref.shape[-1]
    c_out = v_ref.shape[-1]
    locs = locs_ref[0]
    h = jnp.sin(locs @ w1_ref[...])
    h = jnp.sin(h @ w2_ref[...])
    filt = h @ w3_ref[...]
    r2 = jnp.sum(locs * locs, axis=1, keepdims=True)
    bump = jnp.exp(1.0 - 1.0 / (1.0 - _DECAY * r2 * r2))
    a = bump * qw_ref[0]
    qi = jax.lax.broadcasted_iota(jnp.int32, (c_in, c_in * c_out), 0)
    qj = jax.lax.broadcasted_iota(jnp.int32, (c_in, c_in * c_out), 1)
    q_mat = (qj // c_out == qi).astype(jnp.float32)
    rj = jax.lax.broadcasted_iota(jnp.int32, (c_in * c_out, c_out), 0)
    rt = jax.lax.broadcasted_iota(jnp.int32, (c_in * c_out, c_out), 1)
    r_mat = (rj % c_out == rt).astype(jnp.float32)
    for b in range(g_ref.shape[0]):
        g = g_ref[b, 0, 0]
        p = (g @ q_mat) * filt
        v_ref[b, 0, 0] = (p @ r_mat) * a


def _dense_stage(locs_r, qw_r, g_r, w1, w2, w3, n_co, n_sb):
    b = g_r.shape[0]
    s = _S_CHUNK
    c_in = g_r.shape[-1]
    c_out = w3.shape[1] // c_in
    grid = (n_co, n_sb)
    return pl.pallas_call(
        _tc_body,
        grid=grid,
        in_specs=[
            pl.BlockSpec((1, s, 2), lambda co, sb: (co * n_sb + sb, 0, 0)),
            pl.BlockSpec((1, s, 1), lambda co, sb: (co * n_sb + sb, 0, 0)),
            pl.BlockSpec((b, 1, 1, s, c_in), lambda co, sb: (0, co, sb, 0, 0)),
            pl.BlockSpec(w1.shape, lambda co, sb: (0, 0)),
            pl.BlockSpec(w2.shape, lambda co, sb: (0, 0)),
            pl.BlockSpec(w3.shape, lambda co, sb: (0, 0)),
        ],
        out_specs=pl.BlockSpec(
            (b, 1, 1, s, c_out), lambda co, sb: (0, co, sb, 0, 0)
        ),
        out_shape=jax.ShapeDtypeStruct((b, n_co, n_sb, s, c_out), jnp.float32),
    )(locs_r, qw_r, g_r, w1, w2, w3)


def kernel(features, eval_locs, eval_indices, quad_weights, W1, W2, W3):
    b, c_in, p_in = features.shape
    nnz = eval_indices.shape[0]
    c_out = W3.shape[1] // c_in
    p_out = p_in
    seg = nnz // c_in
    n_sb = seg // _S_CHUNK
    idx_out = eval_indices[:, 0].astype(jnp.int32)
    idx_in = eval_indices[:, 1].astype(jnp.int32)

    g, qw_g = _sc_gather(features, quad_weights, idx_in)
    g_r = g.reshape(b, c_in, n_sb, _S_CHUNK, c_in)

    locs_r = eval_locs.reshape(c_in * n_sb, _S_CHUNK, 2)
    qw_r = qw_g.reshape(c_in * n_sb, _S_CHUNK, 1)

    v = _dense_stage(locs_r, qw_r, g_r, W1, W2, W3, c_in, n_sb)
    values2 = v.reshape(b, c_out, nnz)

    return _sc_scatter(values2, idx_out, p_out)
